# Initial kernel scaffold; baseline (speedup 1.0000x reference)
import functools

import jax
import jax.numpy as jnp
from jax import lax
from jax.experimental import pallas as pl
from jax.experimental.pallas import tpu as pltpu
from jax.experimental.pallas import tpu_sc as plsc

B = 4096
L = 50
D = 64
NW = 32
BPW = B // NW
CH = 8
NCHUNK = BPW // CH
IPC = CH * L
NG = 5
GI = IPC // NG


def _pool_kernel(ids_hbm, w_hbm, out_hbm, ids_v, buf, inv_v, pooled, sem):
    wid = lax.axis_index("s") * 2 + lax.axis_index("c")
    pltpu.sync_copy(ids_hbm.at[pl.ds(wid * (BPW * L), BPW * L)], ids_v)

    lane = lax.iota(jnp.int32, 16)
    for g in range(BPW // 16):
        off = (lane + g * 16) * L

        def cnt_body(l, cnt):
            v = plsc.load_gather(ids_v, [off + l])
            return cnt + jnp.where(v != 0, 1.0, 0.0).astype(jnp.float32)

        cnt = lax.fori_loop(0, L, cnt_body, jnp.zeros((16,), jnp.float32))
        inv_v[pl.ds(g * 16, 16)] = 1.0 / (cnt + 1e-8)

    def chunk_body(c, _):
        copies = []
        for g in range(NG):
            copies.append(
                pltpu.async_copy(
                    w_hbm.at[ids_v.at[pl.ds(c * IPC + g * GI, GI)]],
                    buf.at[pl.ds(g * GI, GI)],
                    sem,
                )
            )
        for cp in copies:
            cp.wait()

        for j in range(CH):
            bb = c * CH + j
            accs = [buf[j * L, pl.ds(d * 16, 16)] for d in range(4)]
            for l in range(1, L):
                for d in range(4):
                    accs[d] = accs[d] + buf[j * L + l, pl.ds(d * 16, 16)]
            iv = plsc.load_gather(inv_v, [jnp.full((16,), 0, jnp.int32) + bb])
            for d in range(4):
                pooled[pl.ds(bb * D + d * 16, 16)] = accs[d] * iv
        return 0

    lax.fori_loop(0, NCHUNK, chunk_body, 0)
    pltpu.sync_copy(pooled, out_hbm.at[pl.ds(wid * (BPW * D), BPW * D)])


def kernel(ids, weight):
    ids_flat = ids.reshape(B * L).astype(jnp.int32)
    weight = weight.astype(jnp.float32)
    mesh = plsc.VectorSubcoreMesh(core_axis_name="c", subcore_axis_name="s")
    run = functools.partial(
        pl.kernel,
        mesh=mesh,
        out_type=jax.ShapeDtypeStruct((B * D,), jnp.float32),
        scratch_types=[
            pltpu.VMEM((BPW * L,), jnp.int32),
            pltpu.VMEM((IPC, D), jnp.float32),
            pltpu.VMEM((BPW,), jnp.float32),
            pltpu.VMEM((BPW * D,), jnp.float32),
            pltpu.SemaphoreType.DMA,
        ],
    )(_pool_kernel)
    out_flat = run(ids_flat, weight)
    return out_flat.reshape(B, D)


# speedup vs baseline: 6.6979x; 6.6979x over previous
import functools

import jax
import jax.numpy as jnp
from jax import lax
from jax.experimental import pallas as pl
from jax.experimental.pallas import tpu as pltpu
from jax.experimental.pallas import tpu_sc as plsc

B = 4096
L = 50
D = 64
NW = 32
BPW = B // NW
CH = 8
NCHUNK = BPW // CH
IPC = CH * L
NG = 5
GI = IPC // NG


def _pool_kernel(ids_hbm, w_hbm, out_hbm, ids_v, buf, pooled, sem):
    wid = lax.axis_index("s") * 2 + lax.axis_index("c")
    pltpu.sync_copy(ids_hbm.at[pl.ds(wid * (BPW * L), BPW * L)], ids_v)

    lane = lax.iota(jnp.int32, 16)
    tail_mask = lane >= 14

    def chunk_body(c, _):
        copies = []
        for g in range(NG):
            copies.append(
                pltpu.async_copy(
                    w_hbm.at[ids_v.at[pl.ds(c * IPC + g * GI, GI)]],
                    buf.at[pl.ds(g * GI, GI)],
                    sem,
                )
            )
        for cp in copies:
            cp.wait()

        for j in range(CH):
            bb = c * CH + j
            base = bb * L
            ones = jnp.float32(1.0)
            zero = jnp.float32(0.0)
            s = jnp.where(ids_v[pl.ds(base, 16)] != 0, ones, zero)
            s = s + jnp.where(ids_v[pl.ds(base + 16, 16)] != 0, ones, zero)
            s = s + jnp.where(ids_v[pl.ds(base + 32, 16)] != 0, ones, zero)
            tail = ids_v[pl.ds(base + 34, 16)]
            s = s + jnp.where((tail != 0) & tail_mask, ones, zero)
            cnt = jnp.full((16,), jnp.sum(s))
            inv = 1.0 / (cnt + 1e-8)

            accs = [buf[j * L, pl.ds(d * 16, 16)] for d in range(4)]
            for l in range(1, L):
                for d in range(4):
                    accs[d] = accs[d] + buf[j * L + l, pl.ds(d * 16, 16)]
            for d in range(4):
                pooled[pl.ds(bb * D + d * 16, 16)] = accs[d] * inv
        return 0

    lax.fori_loop(0, NCHUNK, chunk_body, 0)
    pltpu.sync_copy(pooled, out_hbm.at[pl.ds(wid * (BPW * D), BPW * D)])


def kernel(ids, weight):
    ids_flat = ids.reshape(B * L).astype(jnp.int32)
    weight = weight.astype(jnp.float32)
    mesh = plsc.VectorSubcoreMesh(core_axis_name="c", subcore_axis_name="s")
    run = functools.partial(
        pl.kernel,
        mesh=mesh,
        compiler_params=pltpu.CompilerParams(
            needs_layout_passes=False, use_tc_tiling_on_sc=False
        ),
        out_type=jax.ShapeDtypeStruct((B * D,), jnp.float32),
        scratch_types=[
            pltpu.VMEM((BPW * L,), jnp.int32),
            pltpu.VMEM((IPC, D), jnp.float32),
            pltpu.VMEM((BPW * D,), jnp.float32),
            pltpu.SemaphoreType.DMA,
        ],
    )(_pool_kernel)
    out_flat = run(ids_flat, weight)
    return out_flat.reshape(B, D)


# speedup vs baseline: 10.2289x; 1.5272x over previous
import functools

import jax
import jax.numpy as jnp
from jax import lax
from jax.experimental import pallas as pl
from jax.experimental.pallas import tpu as pltpu
from jax.experimental.pallas import tpu_sc as plsc

B = 4096
L = 50
D = 64
NW = 32
BPW = B // NW


def _pool_kernel(ids3_hbm, w_hbm, out_hbm, ids_tv, acc, inv_v, sem):
    wid = lax.axis_index("s") * 2 + lax.axis_index("c")
    pltpu.sync_copy(ids3_hbm.at[wid], ids_tv)

    zero = jnp.zeros((16,), jnp.float32)

    def zero_body(b, _):
        for d in range(4):
            acc[b, pl.ds(d * 16, 16)] = zero
        return 0

    lax.fori_loop(0, BPW, zero_body, 0)

    def fire_body(l, _):
        pltpu.async_copy(w_hbm.at[ids_tv.at[l]], acc, sem, add=True)
        return 0

    lax.fori_loop(0, L, fire_body, 0)

    for g in range(BPW // 16):
        def cnt_body(l, cnt):
            v = ids_tv[l, pl.ds(g * 16, 16)]
            return cnt + jnp.where(v != 0, 1.0, 0.0).astype(jnp.float32)

        cnt = lax.fori_loop(0, L, cnt_body, jnp.zeros((16,), jnp.float32))
        inv_v[pl.ds(g * 16, 16)] = 1.0 / (cnt + 1e-8)

    def drain_body(l, _):
        pltpu.make_async_copy(w_hbm.at[ids_tv.at[0]], acc, sem).wait()
        return 0

    lax.fori_loop(0, L, drain_body, 0)

    def scale_body(b, _):
        iv = jnp.full((16,), inv_v[pl.ds(b, 16)][0])
        for d in range(4):
            acc[b, pl.ds(d * 16, 16)] = acc[b, pl.ds(d * 16, 16)] * iv
        return 0

    lax.fori_loop(0, BPW, scale_body, 0)
    pltpu.sync_copy(acc, out_hbm.at[pl.ds(wid * BPW, BPW)])


def kernel(ids, weight):
    ids3 = ids.astype(jnp.int32).T.reshape(L, NW, BPW).swapaxes(0, 1)
    weight = weight.astype(jnp.float32)
    mesh = plsc.VectorSubcoreMesh(core_axis_name="c", subcore_axis_name="s")
    run = functools.partial(
        pl.kernel,
        mesh=mesh,
        compiler_params=pltpu.CompilerParams(
            needs_layout_passes=False, use_tc_tiling_on_sc=False
        ),
        out_type=jax.ShapeDtypeStruct((B, D), jnp.float32),
        scratch_types=[
            pltpu.VMEM((L, BPW), jnp.int32),
            pltpu.VMEM((BPW, D), jnp.float32),
            pltpu.VMEM((BPW + 16,), jnp.float32),
            pltpu.SemaphoreType.DMA,
        ],
    )(_pool_kernel)
    return run(ids3, weight)


# speedup vs baseline: 10.2344x; 1.0005x over previous
import functools

import jax
import jax.numpy as jnp
from jax import lax
from jax.experimental import pallas as pl
from jax.experimental.pallas import tpu as pltpu
from jax.experimental.pallas import tpu_sc as plsc

B = 4096
L = 50
D = 64
NW = 32
BPW = B // NW


def _pool_kernel(ids_hbm, w_hbm, out_hbm, ids_v, ids_tv, acc, inv_v, sem):
    wid = lax.axis_index("s") * 2 + lax.axis_index("c")
    pltpu.sync_copy(ids_hbm.at[pl.ds(wid * (BPW * L), BPW * L)], ids_v)

    zero = jnp.zeros((16,), jnp.float32)

    def zero_body(b, _):
        for d in range(4):
            acc[b, pl.ds(d * 16, 16)] = zero
        return 0

    lax.fori_loop(0, BPW, zero_body, 0)

    lane = lax.iota(jnp.int32, 16)

    def fire_body(l, _):
        for g in range(BPW // 16):
            idx = (lane + g * 16) * L + l
            ids_tv[l, pl.ds(g * 16, 16)] = plsc.load_gather(ids_v, [idx])
        pltpu.async_copy(w_hbm.at[ids_tv.at[l]], acc, sem, add=True)
        return 0

    lax.fori_loop(0, L, fire_body, 0)

    for g in range(BPW // 16):
        def cnt_body(l, cnt):
            v = ids_tv[l, pl.ds(g * 16, 16)]
            return cnt + jnp.where(v != 0, 1.0, 0.0).astype(jnp.float32)

        cnt = lax.fori_loop(0, L, cnt_body, jnp.zeros((16,), jnp.float32))
        inv_v[pl.ds(g * 16, 16)] = 1.0 / (cnt + 1e-8)

    def drain_body(l, _):
        pltpu.make_async_copy(w_hbm.at[ids_tv.at[0]], acc, sem).wait()
        return 0

    lax.fori_loop(0, L, drain_body, 0)

    def scale_body(b, _):
        iv = jnp.full((16,), inv_v[pl.ds(b, 16)][0])
        for d in range(4):
            acc[b, pl.ds(d * 16, 16)] = acc[b, pl.ds(d * 16, 16)] * iv
        return 0

    lax.fori_loop(0, BPW, scale_body, 0)
    pltpu.sync_copy(acc, out_hbm.at[pl.ds(wid * BPW, BPW)])


def kernel(ids, weight):
    ids_flat = ids.astype(jnp.int32).reshape(B * L)
    weight = weight.astype(jnp.float32)
    mesh = plsc.VectorSubcoreMesh(core_axis_name="c", subcore_axis_name="s")
    run = functools.partial(
        pl.kernel,
        mesh=mesh,
        compiler_params=pltpu.CompilerParams(
            needs_layout_passes=False, use_tc_tiling_on_sc=False
        ),
        out_type=jax.ShapeDtypeStruct((B, D), jnp.float32),
        scratch_types=[
            pltpu.VMEM((BPW * L,), jnp.int32),
            pltpu.VMEM((L, BPW), jnp.int32),
            pltpu.VMEM((BPW, D), jnp.float32),
            pltpu.VMEM((BPW + 16,), jnp.float32),
            pltpu.SemaphoreType.DMA,
        ],
    )(_pool_kernel)
    return run(ids_flat, weight)
